# baseline (speedup 1.0000x reference)
import functools

import jax
import jax.numpy as jnp
from jax import lax
from jax.experimental import pallas as pl
from jax.experimental.pallas import tpu as pltpu
from jax.experimental.pallas import tpu_sc as plsc


def _table_body(emb_ref, w_ref, b_ref, tab_ref):
    tab_ref[...] = lax.dot_general(
        emb_ref[...], w_ref[...],
        dimension_numbers=(((1,), (1,)), ((), ())),
        preferred_element_type=jnp.float32,
    ) + b_ref[...]


def _make_table(emb, w, b):
    V = w.shape[0]
    return pl.pallas_call(
        _table_body,
        out_shape=jax.ShapeDtypeStruct((emb.shape[0], V), jnp.float32),
    )(emb, w, b.reshape(1, V))


def _sc_gather(table, ids_flat):
    T = ids_flat.shape[0]
    V = table.shape[1]
    info = plsc.get_sparse_core_info()
    nc, ns = info.num_cores, info.num_subcores
    nw = nc * ns
    tpw = T // nw
    CH = 40
    nch = tpw // CH
    mesh = plsc.VectorSubcoreMesh(core_axis_name="c", subcore_axis_name="s")

    @functools.partial(
        pl.kernel, mesh=mesh,
        compiler_params=pltpu.CompilerParams(use_tc_tiling_on_sc=False),
        out_type=jax.ShapeDtypeStruct((T, V), jnp.float32),
        scratch_types=[
            pltpu.VMEM((tpw,), jnp.int32),
            pltpu.VMEM((CH, V), jnp.float32),
            pltpu.VMEM((CH, V), jnp.float32),
            pltpu.SemaphoreType.DMA,
            pltpu.SemaphoreType.DMA,
            pltpu.SemaphoreType.DMA,
            pltpu.SemaphoreType.DMA,
        ],
    )
    def k(tab_hbm, ids_hbm, out_hbm, idx_v, buf_a, buf_b, sga, sgb, soa, sob):
        wid = lax.axis_index("s") * nc + lax.axis_index("c")
        base = wid * tpw
        pltpu.sync_copy(ids_hbm.at[pl.ds(base, tpw)], idx_v)

        def g_start(c, buf, sem):
            idx = idx_v.at[pl.ds(c * CH, CH)]
            pltpu.make_async_copy(tab_hbm.at[idx], buf, sem).start()

        def g_wait(c, buf, sem):
            idx = idx_v.at[pl.ds(c * CH, CH)]
            pltpu.make_async_copy(tab_hbm.at[idx], buf, sem).wait()

        def o_start(c, buf, sem):
            pltpu.make_async_copy(
                buf, out_hbm.at[pl.ds(base + c * CH, CH)], sem).start()

        def o_wait(c, buf, sem):
            pltpu.make_async_copy(
                buf, out_hbm.at[pl.ds(base + c * CH, CH)], sem).wait()

        g_start(0, buf_a, sga)

        def body(i, carry):
            c = 2 * i
            g_wait(c, buf_a, sga)

            @pl.when(i > 0)
            def _():
                o_wait(c - 1, buf_b, sob)

            g_start(c + 1, buf_b, sgb)
            o_start(c, buf_a, soa)
            g_wait(c + 1, buf_b, sgb)
            o_wait(c, buf_a, soa)

            @pl.when(i < nch // 2 - 1)
            def _():
                g_start(c + 2, buf_a, sga)

            o_start(c + 1, buf_b, sob)
            return carry

        lax.fori_loop(0, nch // 2, body, 0)
        o_wait(nch - 1, buf_b, sob)

    return k(table, ids_flat)


def kernel(input_ids, emb_table, lm_head_w, lm_head_b):
    B, L = input_ids.shape
    V = lm_head_w.shape[0]
    table = _make_table(emb_table, lm_head_w, lm_head_b)
    out = _sc_gather(table, input_ids.reshape(B * L))
    return out.reshape(B, L, V)
